# baseline (speedup 1.0000x reference)
import functools

import jax
import jax.numpy as jnp
from jax import lax
from jax.experimental import pallas as pl
from jax.experimental.pallas import tpu as pltpu
from jax.experimental.pallas import tpu_sc as plsc

D_MODEL = 128
MAX_REL = 128
SEQ = 1024
NC, NS, L = 2, 16, 16
NW = NC * NS
ROWS_PER_W = SEQ // NW
HALF = SEQ // 2
WIN = HALF + ROWS_PER_W - 1
GCH = 128
NG = -(-WIN // GCH)
NBUF = 4


def _rel_pos_body(emb_hbm, out_hbm, idx_v, win_v, gsem, osem):
    c = lax.axis_index("c")
    s = lax.axis_index("s")
    w = s * NC + c
    i0 = w * ROWS_PER_W

    for h in range(2):
        base = (SEQ - ROWS_PER_W) - i0 + HALF * h
        for t in range(NG * GCH // L):
            v = base + t * L + lax.iota(jnp.int32, L)
            pos = jnp.clip((SEQ - 1) - v, -MAX_REL, MAX_REL) + MAX_REL
            idx_v[0, pl.ds(t * L, L)] = pos
        gathers = [
            pltpu.async_copy(
                emb_hbm.at[idx_v.at[0, pl.ds(g * GCH, GCH)]],
                win_v.at[pl.ds(g * GCH, GCH)],
                gsem,
            )
            for g in range(NG)
        ]
        for d in gathers:
            d.wait()

        pending = []
        for r in range(ROWS_PER_W):
            pending.append(
                pltpu.async_copy(
                    win_v.at[pl.ds((ROWS_PER_W - 1) - r, HALF)],
                    out_hbm.at[i0 + r, pl.ds(HALF * h, HALF)],
                    osem,
                )
            )
            if len(pending) >= NBUF:
                pending.pop(0).wait()
        for d in pending:
            d.wait()


@jax.jit
def _rel_pos_sc(embeddings):
    mesh = plsc.VectorSubcoreMesh(
        core_axis_name="c", subcore_axis_name="s",
        num_cores=NC, num_subcores=NS,
    )
    return pl.kernel(
        _rel_pos_body,
        out_type=jax.ShapeDtypeStruct((SEQ, SEQ, D_MODEL), jnp.float32),
        mesh=mesh,
        scratch_types=[
            pltpu.VMEM((1, NG * GCH), jnp.int32),
            pltpu.VMEM((NG * GCH, D_MODEL), jnp.float32),
            pltpu.SemaphoreType.DMA,
            pltpu.SemaphoreType.DMA,
        ],
    )(embeddings)


def kernel(embeddings, seq_len):
    del seq_len
    return _rel_pos_sc(embeddings)


# speedup vs baseline: 4.1717x; 4.1717x over previous
import functools

import jax
import jax.numpy as jnp
from jax import lax
from jax.experimental import pallas as pl
from jax.experimental.pallas import tpu as pltpu
from jax.experimental.pallas import tpu_sc as plsc

D_MODEL = 128
MAX_REL = 128
SEQ = 1024
RPAD = 2 * SEQ
NC, NS, L = 2, 16, 16
NW = NC * NS
FILL = RPAD // NS
ROWS_PER_W = SEQ // NW
HALF = SEQ // 2
K_S = 13
K_D = ROWS_PER_W - K_S
WIN = HALF + K_S - 1
NBUF = 8


def _rel_pos_body(emb_hbm, out_hbm, idx_v, rows_v, win_v, r_sh, gsem, dsem, ssem):
    c = lax.axis_index("c")
    s = lax.axis_index("s")

    base = s * FILL
    for t in range(FILL // L):
        v = base + t * L + lax.iota(jnp.int32, L)
        pos = jnp.clip((SEQ - 1) - v, -MAX_REL, MAX_REL) + MAX_REL
        idx_v[pl.ds(t * L, L)] = pos
    pltpu.async_copy(emb_hbm.at[idx_v], rows_v, gsem).wait()
    pltpu.sync_copy(rows_v, r_sh.at[pl.ds(base, FILL)])
    plsc.subcore_barrier()

    w = s * NC + c
    i0 = w * ROWS_PER_W

    wbase = (SEQ - ROWS_PER_W) - i0
    pltpu.sync_copy(r_sh.at[pl.ds(wbase, WIN)], win_v)
    streams = [
        pltpu.async_copy(
            win_v.at[pl.ds((K_S - 1) - r, HALF)],
            out_hbm.at[i0 + K_D + r, pl.ds(0, HALF)],
            ssem,
        )
        for r in range(K_S)
    ]

    pending = []

    def fire(src_off, dst_i, dst_j, n):
        pending.append(
            pltpu.async_copy(
                r_sh.at[pl.ds(src_off, n)],
                out_hbm.at[dst_i, pl.ds(dst_j, n)],
                dsem,
            )
        )
        if len(pending) >= NBUF:
            pending.pop(0).wait()

    for r in range(K_D):
        i = i0 + r
        fire((SEQ - 1) - i, i, 0, SEQ)
    for r in range(K_S):
        i = i0 + K_D + r
        fire((SEQ - 1) - i + HALF, i, HALF, HALF)
    for d in pending:
        d.wait()
    for d in streams:
        d.wait()


@jax.jit
def _rel_pos_sc(embeddings):
    mesh = plsc.VectorSubcoreMesh(
        core_axis_name="c", subcore_axis_name="s",
        num_cores=NC, num_subcores=NS,
    )
    return pl.kernel(
        _rel_pos_body,
        out_type=jax.ShapeDtypeStruct((SEQ, SEQ, D_MODEL), jnp.float32),
        mesh=mesh,
        scratch_types=[
            pltpu.VMEM((FILL,), jnp.int32),
            pltpu.VMEM((FILL, D_MODEL), jnp.float32),
            pltpu.VMEM((WIN, D_MODEL), jnp.float32),
            pltpu.VMEM_SHARED((RPAD, D_MODEL), jnp.float32),
            pltpu.SemaphoreType.DMA,
            pltpu.SemaphoreType.DMA,
            pltpu.SemaphoreType.DMA,
        ],
    )(embeddings)


def kernel(embeddings, seq_len):
    del seq_len
    return _rel_pos_sc(embeddings)


# speedup vs baseline: 4.2938x; 1.0293x over previous
import functools

import jax
import jax.numpy as jnp
from jax import lax
from jax.experimental import pallas as pl
from jax.experimental.pallas import tpu as pltpu
from jax.experimental.pallas import tpu_sc as plsc

D_MODEL = 128
MAX_REL = 128
SEQ = 1024
RPAD = 2 * SEQ
NC, NS, L = 2, 16, 16
NW = NC * NS
FILL = RPAD // NS
ROWS_PER_W = SEQ // NW
HALF = SEQ // 2
K_S = 16
K_D = ROWS_PER_W - K_S
WIN = HALF + K_S - 1
NBUF = 8


def _rel_pos_body(emb_hbm, out_hbm, idx_v, rows_v, win_v, r_sh, gsem, dsem, ssem):
    c = lax.axis_index("c")
    s = lax.axis_index("s")

    base = s * FILL
    for t in range(FILL // L):
        v = base + t * L + lax.iota(jnp.int32, L)
        pos = jnp.clip((SEQ - 1) - v, -MAX_REL, MAX_REL) + MAX_REL
        idx_v[pl.ds(t * L, L)] = pos
    pltpu.async_copy(emb_hbm.at[idx_v], rows_v, gsem).wait()
    pltpu.sync_copy(rows_v, r_sh.at[pl.ds(base, FILL)])
    plsc.subcore_barrier()

    w = s * NC + c
    i0 = w * ROWS_PER_W

    wbase = (SEQ - ROWS_PER_W) - i0
    pltpu.sync_copy(r_sh.at[pl.ds(wbase, WIN)], win_v)
    streams = [
        pltpu.async_copy(
            win_v.at[pl.ds((K_S - 1) - r, HALF)],
            out_hbm.at[i0 + K_D + r, pl.ds(0, HALF)],
            ssem,
        )
        for r in range(K_S)
    ]

    pending = []

    def fire(src_off, dst_i, dst_j, n):
        pending.append(
            pltpu.async_copy(
                r_sh.at[pl.ds(src_off, n)],
                out_hbm.at[dst_i, pl.ds(dst_j, n)],
                dsem,
            )
        )
        if len(pending) >= NBUF:
            pending.pop(0).wait()

    for r in range(K_D):
        i = i0 + r
        fire((SEQ - 1) - i, i, 0, SEQ)
    for r in range(K_S):
        i = i0 + K_D + r
        fire((SEQ - 1) - i + HALF, i, HALF, HALF)
    for d in pending:
        d.wait()
    for d in streams:
        d.wait()


@jax.jit
def _rel_pos_sc(embeddings):
    mesh = plsc.VectorSubcoreMesh(
        core_axis_name="c", subcore_axis_name="s",
        num_cores=NC, num_subcores=NS,
    )
    return pl.kernel(
        _rel_pos_body,
        out_type=jax.ShapeDtypeStruct((SEQ, SEQ, D_MODEL), jnp.float32),
        mesh=mesh,
        scratch_types=[
            pltpu.VMEM((FILL,), jnp.int32),
            pltpu.VMEM((FILL, D_MODEL), jnp.float32),
            pltpu.VMEM((WIN, D_MODEL), jnp.float32),
            pltpu.VMEM_SHARED((RPAD, D_MODEL), jnp.float32),
            pltpu.SemaphoreType.DMA,
            pltpu.SemaphoreType.DMA,
            pltpu.SemaphoreType.DMA,
        ],
    )(embeddings)


def kernel(embeddings, seq_len):
    del seq_len
    return _rel_pos_sc(embeddings)


# speedup vs baseline: 4.4640x; 1.0396x over previous
import functools

import jax
import jax.numpy as jnp
from jax import lax
from jax.experimental import pallas as pl
from jax.experimental.pallas import tpu as pltpu
from jax.experimental.pallas import tpu_sc as plsc

D_MODEL = 128
MAX_REL = 128
SEQ = 1024
RPAD = 2 * SEQ
NC, NS, L = 2, 16, 16
NW = NC * NS
FILL = RPAD // NS
ROWS_PER_W = SEQ // NW
HALF = SEQ // 2
K_S = 20
K_D = ROWS_PER_W - K_S
WIN = HALF + K_S - 1
NBUF = 8


def _rel_pos_body(emb_hbm, out_hbm, idx_v, rows_v, win_v, r_sh, gsem, dsem, ssem):
    c = lax.axis_index("c")
    s = lax.axis_index("s")

    base = s * FILL
    for t in range(FILL // L):
        v = base + t * L + lax.iota(jnp.int32, L)
        pos = jnp.clip((SEQ - 1) - v, -MAX_REL, MAX_REL) + MAX_REL
        idx_v[pl.ds(t * L, L)] = pos
    pltpu.async_copy(emb_hbm.at[idx_v], rows_v, gsem).wait()
    pltpu.sync_copy(rows_v, r_sh.at[pl.ds(base, FILL)])
    plsc.subcore_barrier()

    w = s * NC + c
    i0 = w * ROWS_PER_W

    wbase = (SEQ - ROWS_PER_W) - i0
    pltpu.sync_copy(r_sh.at[pl.ds(wbase, WIN)], win_v)
    streams = [
        pltpu.async_copy(
            win_v.at[pl.ds((K_S - 1) - r, HALF)],
            out_hbm.at[i0 + K_D + r, pl.ds(0, HALF)],
            ssem,
        )
        for r in range(K_S)
    ]

    pending = []

    def fire(src_off, dst_i, dst_j, n):
        pending.append(
            pltpu.async_copy(
                r_sh.at[pl.ds(src_off, n)],
                out_hbm.at[dst_i, pl.ds(dst_j, n)],
                dsem,
            )
        )
        if len(pending) >= NBUF:
            pending.pop(0).wait()

    for r in range(K_D):
        i = i0 + r
        fire((SEQ - 1) - i, i, 0, SEQ)
    for r in range(K_S):
        i = i0 + K_D + r
        fire((SEQ - 1) - i + HALF, i, HALF, HALF)
    for d in pending:
        d.wait()
    for d in streams:
        d.wait()


@jax.jit
def _rel_pos_sc(embeddings):
    mesh = plsc.VectorSubcoreMesh(
        core_axis_name="c", subcore_axis_name="s",
        num_cores=NC, num_subcores=NS,
    )
    return pl.kernel(
        _rel_pos_body,
        out_type=jax.ShapeDtypeStruct((SEQ, SEQ, D_MODEL), jnp.float32),
        mesh=mesh,
        scratch_types=[
            pltpu.VMEM((FILL,), jnp.int32),
            pltpu.VMEM((FILL, D_MODEL), jnp.float32),
            pltpu.VMEM((WIN, D_MODEL), jnp.float32),
            pltpu.VMEM_SHARED((RPAD, D_MODEL), jnp.float32),
            pltpu.SemaphoreType.DMA,
            pltpu.SemaphoreType.DMA,
            pltpu.SemaphoreType.DMA,
        ],
    )(embeddings)


def kernel(embeddings, seq_len):
    del seq_len
    return _rel_pos_sc(embeddings)


# speedup vs baseline: 5.6804x; 1.2725x over previous
import functools

import jax
import jax.numpy as jnp
from jax import lax
from jax.experimental import pallas as pl
from jax.experimental.pallas import tpu as pltpu
from jax.experimental.pallas import tpu_sc as plsc

D_MODEL = 128
MAX_REL = 128
SEQ = 1024
RPAD = 2 * SEQ
NC, NS, L = 2, 16, 16
NW = NC * NS
FILL = RPAD // NS
ROWS_PER_W = SEQ // NW
HALF = SEQ // 2
K_S = 20
K_D = ROWS_PER_W - K_S
WIN = HALF + K_S - 1
NBUF = 8
EMBV = FILL + 8
EPAD = 264


def _rel_pos_body(emb_hbm, out_hbm, emb_v, rows_v, win_v, r_sh, dsem, ssem):
    c = lax.axis_index("c")
    s = lax.axis_index("s")

    base = s * FILL
    src_min = jnp.clip((SEQ - 1) - (base + FILL - 1), -MAX_REL, MAX_REL) + MAX_REL
    start = jnp.minimum((src_min // 8) * 8, MAX_REL)
    pltpu.sync_copy(emb_hbm.at[pl.ds(start, EMBV)], emb_v)

    def fill_row(t, _):
        src = jnp.clip((SEQ - 1) - (base + t), -MAX_REL, MAX_REL) + MAX_REL
        for k in range(D_MODEL // L):
            rows_v[t, pl.ds(k * L, L)] = emb_v[src - start, pl.ds(k * L, L)]
        return 0

    lax.fori_loop(0, FILL, fill_row, 0)
    pltpu.sync_copy(rows_v, r_sh.at[pl.ds(base, FILL)])
    plsc.subcore_barrier()

    w = s * NC + c
    i0 = w * ROWS_PER_W

    wbase = (SEQ - ROWS_PER_W) - i0
    pltpu.sync_copy(r_sh.at[pl.ds(wbase, WIN)], win_v)
    streams = [
        pltpu.async_copy(
            win_v.at[pl.ds((K_S - 1) - r, HALF)],
            out_hbm.at[i0 + K_D + r, pl.ds(0, HALF)],
            ssem,
        )
        for r in range(K_S)
    ]

    pending = []

    def fire(src_off, dst_i, dst_j, n):
        pending.append(
            pltpu.async_copy(
                r_sh.at[pl.ds(src_off, n)],
                out_hbm.at[dst_i, pl.ds(dst_j, n)],
                dsem,
            )
        )
        if len(pending) >= NBUF:
            pending.pop(0).wait()

    for r in range(K_D):
        i = i0 + r
        fire((SEQ - 1) - i, i, 0, SEQ)
    for r in range(K_S):
        i = i0 + K_D + r
        fire((SEQ - 1) - i + HALF, i, HALF, HALF)
    for d in pending:
        d.wait()
    for d in streams:
        d.wait()


@jax.jit
def _rel_pos_sc(embeddings):
    mesh = plsc.VectorSubcoreMesh(
        core_axis_name="c", subcore_axis_name="s",
        num_cores=NC, num_subcores=NS,
    )
    return pl.kernel(
        _rel_pos_body,
        out_type=jax.ShapeDtypeStruct((SEQ, SEQ, D_MODEL), jnp.float32),
        mesh=mesh,
        scratch_types=[
            pltpu.VMEM((EMBV, D_MODEL), jnp.float32),
            pltpu.VMEM((FILL, D_MODEL), jnp.float32),
            pltpu.VMEM((WIN, D_MODEL), jnp.float32),
            pltpu.VMEM_SHARED((RPAD, D_MODEL), jnp.float32),
            pltpu.SemaphoreType.DMA,
            pltpu.SemaphoreType.DMA,
        ],
    )(embeddings)


def kernel(embeddings, seq_len):
    del seq_len
    emb_pad = jnp.pad(embeddings, ((0, EPAD - embeddings.shape[0]), (0, 0)))
    return _rel_pos_sc(emb_pad)


# speedup vs baseline: 5.9948x; 1.0553x over previous
import functools

import jax
import jax.numpy as jnp
from jax import lax
from jax.experimental import pallas as pl
from jax.experimental.pallas import tpu as pltpu
from jax.experimental.pallas import tpu_sc as plsc

D_MODEL = 128
MAX_REL = 128
SEQ = 1024
RPAD = 2 * SEQ
NC, NS, L = 2, 16, 16
NW = NC * NS
FILL = RPAD // NS
ROWS_PER_W = SEQ // NW
HALF = SEQ // 2
K_S = 24
K_D = ROWS_PER_W - K_S
WIN = HALF + K_S - 1
NBUF = 8
EMBV = FILL + 8
EPAD = 264


def _rel_pos_body(emb_hbm, out_hbm, emb_v, rows_v, win_v, r_sh, dsem, ssem):
    c = lax.axis_index("c")
    s = lax.axis_index("s")

    base = s * FILL
    src_min = jnp.clip((SEQ - 1) - (base + FILL - 1), -MAX_REL, MAX_REL) + MAX_REL
    start = jnp.minimum((src_min // 8) * 8, MAX_REL)
    pltpu.sync_copy(emb_hbm.at[pl.ds(start, EMBV)], emb_v)

    def fill_row(t, _):
        src = jnp.clip((SEQ - 1) - (base + t), -MAX_REL, MAX_REL) + MAX_REL
        for k in range(D_MODEL // L):
            rows_v[t, pl.ds(k * L, L)] = emb_v[src - start, pl.ds(k * L, L)]
        return 0

    lax.fori_loop(0, FILL, fill_row, 0)
    pltpu.sync_copy(rows_v, r_sh.at[pl.ds(base, FILL)])
    plsc.subcore_barrier()

    w = s * NC + c
    i0 = w * ROWS_PER_W

    wbase = (SEQ - ROWS_PER_W) - i0
    pltpu.sync_copy(r_sh.at[pl.ds(wbase, WIN)], win_v)
    streams = [
        pltpu.async_copy(
            win_v.at[pl.ds((K_S - 1) - r, HALF)],
            out_hbm.at[i0 + K_D + r, pl.ds(0, HALF)],
            ssem,
        )
        for r in range(K_S)
    ]

    pending = []

    def fire(src_off, dst_i, dst_j, n):
        pending.append(
            pltpu.async_copy(
                r_sh.at[pl.ds(src_off, n)],
                out_hbm.at[dst_i, pl.ds(dst_j, n)],
                dsem,
            )
        )
        if len(pending) >= NBUF:
            pending.pop(0).wait()

    for r in range(K_D):
        i = i0 + r
        fire((SEQ - 1) - i, i, 0, SEQ)
    for r in range(K_S):
        i = i0 + K_D + r
        fire((SEQ - 1) - i + HALF, i, HALF, HALF)
    for d in pending:
        d.wait()
    for d in streams:
        d.wait()


@jax.jit
def _rel_pos_sc(embeddings):
    mesh = plsc.VectorSubcoreMesh(
        core_axis_name="c", subcore_axis_name="s",
        num_cores=NC, num_subcores=NS,
    )
    return pl.kernel(
        _rel_pos_body,
        out_type=jax.ShapeDtypeStruct((SEQ, SEQ, D_MODEL), jnp.float32),
        mesh=mesh,
        scratch_types=[
            pltpu.VMEM((EMBV, D_MODEL), jnp.float32),
            pltpu.VMEM((FILL, D_MODEL), jnp.float32),
            pltpu.VMEM((WIN, D_MODEL), jnp.float32),
            pltpu.VMEM_SHARED((RPAD, D_MODEL), jnp.float32),
            pltpu.SemaphoreType.DMA,
            pltpu.SemaphoreType.DMA,
        ],
    )(embeddings)


def kernel(embeddings, seq_len):
    del seq_len
    emb_pad = jnp.pad(embeddings, ((0, EPAD - embeddings.shape[0]), (0, 0)))
    return _rel_pos_sc(emb_pad)


# speedup vs baseline: 6.3557x; 1.0602x over previous
import functools

import jax
import jax.numpy as jnp
from jax import lax
from jax.experimental import pallas as pl
from jax.experimental.pallas import tpu as pltpu
from jax.experimental.pallas import tpu_sc as plsc

D_MODEL = 128
MAX_REL = 128
SEQ = 1024
RPAD = 2 * SEQ
NC, NS, L = 2, 16, 16
NW = NC * NS
FILL = RPAD // NS
ROWS_PER_W = SEQ // NW
HALF = SEQ // 2
K_S = 28
K_D = ROWS_PER_W - K_S
WIN = HALF + K_S - 1
NBUF = 8
EMBV = FILL + 8
EPAD = 264


def _rel_pos_body(emb_hbm, out_hbm, emb_v, rows_v, win_v, r_sh, dsem, ssem):
    c = lax.axis_index("c")
    s = lax.axis_index("s")

    base = s * FILL
    src_min = jnp.clip((SEQ - 1) - (base + FILL - 1), -MAX_REL, MAX_REL) + MAX_REL
    start = jnp.minimum((src_min // 8) * 8, MAX_REL)
    pltpu.sync_copy(emb_hbm.at[pl.ds(start, EMBV)], emb_v)

    def fill_row(t, _):
        src = jnp.clip((SEQ - 1) - (base + t), -MAX_REL, MAX_REL) + MAX_REL
        for k in range(D_MODEL // L):
            rows_v[t, pl.ds(k * L, L)] = emb_v[src - start, pl.ds(k * L, L)]
        return 0

    lax.fori_loop(0, FILL, fill_row, 0)
    pltpu.sync_copy(rows_v, r_sh.at[pl.ds(base, FILL)])
    plsc.subcore_barrier()

    w = s * NC + c
    i0 = w * ROWS_PER_W

    wbase = (SEQ - ROWS_PER_W) - i0
    pltpu.sync_copy(r_sh.at[pl.ds(wbase, WIN)], win_v)
    streams = [
        pltpu.async_copy(
            win_v.at[pl.ds((K_S - 1) - r, HALF)],
            out_hbm.at[i0 + K_D + r, pl.ds(0, HALF)],
            ssem,
        )
        for r in range(K_S)
    ]

    pending = []

    def fire(src_off, dst_i, dst_j, n):
        pending.append(
            pltpu.async_copy(
                r_sh.at[pl.ds(src_off, n)],
                out_hbm.at[dst_i, pl.ds(dst_j, n)],
                dsem,
            )
        )
        if len(pending) >= NBUF:
            pending.pop(0).wait()

    for r in range(K_D):
        i = i0 + r
        fire((SEQ - 1) - i, i, 0, SEQ)
    for r in range(K_S):
        i = i0 + K_D + r
        fire((SEQ - 1) - i + HALF, i, HALF, HALF)
    for d in pending:
        d.wait()
    for d in streams:
        d.wait()


@jax.jit
def _rel_pos_sc(embeddings):
    mesh = plsc.VectorSubcoreMesh(
        core_axis_name="c", subcore_axis_name="s",
        num_cores=NC, num_subcores=NS,
    )
    return pl.kernel(
        _rel_pos_body,
        out_type=jax.ShapeDtypeStruct((SEQ, SEQ, D_MODEL), jnp.float32),
        mesh=mesh,
        scratch_types=[
            pltpu.VMEM((EMBV, D_MODEL), jnp.float32),
            pltpu.VMEM((FILL, D_MODEL), jnp.float32),
            pltpu.VMEM((WIN, D_MODEL), jnp.float32),
            pltpu.VMEM_SHARED((RPAD, D_MODEL), jnp.float32),
            pltpu.SemaphoreType.DMA,
            pltpu.SemaphoreType.DMA,
        ],
    )(embeddings)


def kernel(embeddings, seq_len):
    del seq_len
    emb_pad = jnp.pad(embeddings, ((0, EPAD - embeddings.shape[0]), (0, 0)))
    return _rel_pos_sc(emb_pad)


# speedup vs baseline: 6.7595x; 1.0635x over previous
import functools

import jax
import jax.numpy as jnp
from jax import lax
from jax.experimental import pallas as pl
from jax.experimental.pallas import tpu as pltpu
from jax.experimental.pallas import tpu_sc as plsc

D_MODEL = 128
MAX_REL = 128
SEQ = 1024
RPAD = 2 * SEQ
NC, NS, L = 2, 16, 16
NW = NC * NS
FILL = RPAD // NS
ROWS_PER_W = SEQ // NW
HALF = SEQ // 2
K_S = 32
K_D = ROWS_PER_W - K_S
WIN = HALF + K_S - 1
NBUF = 8
EMBV = FILL + 8
EPAD = 264


def _rel_pos_body(emb_hbm, out_hbm, emb_v, rows_v, win_v, r_sh, dsem, ssem):
    c = lax.axis_index("c")
    s = lax.axis_index("s")

    base = s * FILL
    src_min = jnp.clip((SEQ - 1) - (base + FILL - 1), -MAX_REL, MAX_REL) + MAX_REL
    start = jnp.minimum((src_min // 8) * 8, MAX_REL)
    pltpu.sync_copy(emb_hbm.at[pl.ds(start, EMBV)], emb_v)

    def fill_row(t, _):
        src = jnp.clip((SEQ - 1) - (base + t), -MAX_REL, MAX_REL) + MAX_REL
        for k in range(D_MODEL // L):
            rows_v[t, pl.ds(k * L, L)] = emb_v[src - start, pl.ds(k * L, L)]
        return 0

    lax.fori_loop(0, FILL, fill_row, 0)
    pltpu.sync_copy(rows_v, r_sh.at[pl.ds(base, FILL)])
    plsc.subcore_barrier()

    w = s * NC + c
    i0 = w * ROWS_PER_W

    wbase = (SEQ - ROWS_PER_W) - i0
    pltpu.sync_copy(r_sh.at[pl.ds(wbase, WIN)], win_v)
    streams = [
        pltpu.async_copy(
            win_v.at[pl.ds((K_S - 1) - r, HALF)],
            out_hbm.at[i0 + K_D + r, pl.ds(0, HALF)],
            ssem,
        )
        for r in range(K_S)
    ]

    pending = []

    def fire(src_off, dst_i, dst_j, n):
        pending.append(
            pltpu.async_copy(
                r_sh.at[pl.ds(src_off, n)],
                out_hbm.at[dst_i, pl.ds(dst_j, n)],
                dsem,
            )
        )
        if len(pending) >= NBUF:
            pending.pop(0).wait()

    for r in range(K_D):
        i = i0 + r
        fire((SEQ - 1) - i, i, 0, SEQ)
    for r in range(K_S):
        i = i0 + K_D + r
        fire((SEQ - 1) - i + HALF, i, HALF, HALF)
    for d in pending:
        d.wait()
    for d in streams:
        d.wait()


@jax.jit
def _rel_pos_sc(embeddings):
    mesh = plsc.VectorSubcoreMesh(
        core_axis_name="c", subcore_axis_name="s",
        num_cores=NC, num_subcores=NS,
    )
    return pl.kernel(
        _rel_pos_body,
        out_type=jax.ShapeDtypeStruct((SEQ, SEQ, D_MODEL), jnp.float32),
        mesh=mesh,
        scratch_types=[
            pltpu.VMEM((EMBV, D_MODEL), jnp.float32),
            pltpu.VMEM((FILL, D_MODEL), jnp.float32),
            pltpu.VMEM((WIN, D_MODEL), jnp.float32),
            pltpu.VMEM_SHARED((RPAD, D_MODEL), jnp.float32),
            pltpu.SemaphoreType.DMA,
            pltpu.SemaphoreType.DMA,
        ],
    )(embeddings)


def kernel(embeddings, seq_len):
    del seq_len
    emb_pad = jnp.pad(embeddings, ((0, EPAD - embeddings.shape[0]), (0, 0)))
    return _rel_pos_sc(emb_pad)
